# baseline (speedup 1.0000x reference)
import jax
import jax.numpy as jnp
from jax import lax
from jax.experimental import pallas as pl
from jax.experimental.pallas import tpu as pltpu
from jax.experimental.pallas import tpu_sc as plsc

_C = 1000
_TL = 16
_J = 128
_NBEST = 5
_CPS = 64
_CHUNK = 16
_BIGI = 1 << 30
_INF = float("inf")


def _top5_insert(slots, v, c):
    sv, sx = slots
    dup = (c == sx[0]) | (c == sx[1]) | (c == sx[2]) | (c == sx[3]) | (c == sx[4])
    lt = [jnp.logical_and(v < sv[k], jnp.logical_not(dup)) for k in range(5)]
    nv = [jnp.where(lt[0], v, sv[0])]
    nx = [jnp.where(lt[0], c, sx[0])]
    for k in range(1, 5):
        nv.append(jnp.where(lt[k], jnp.where(lt[k - 1], sv[k - 1], v), sv[k]))
        nx.append(jnp.where(lt[k], jnp.where(lt[k - 1], sx[k - 1], c), sx[k]))
    return nv, nx


def _body(w_hbm, b_hbm, ls_hbm, sos_hbm, out_hbm,
          w_v, sos_v, b_v, lab_v, labf_v, p_v, prow_v, rowbuf,
          win_v, io16f, io16i, out_v, cv_v, ci_v, dv_v, di_v,
          cand_s, candi_s, win_s, pbuf_s, dmv_s, dmi_s):
    cid = lax.axis_index("c")
    sid = lax.axis_index("s")
    it = lax.iota(jnp.int32, 16)

    with jax.named_scope("ph0_stage"):
        pltpu.sync_copy(w_hbm, w_v)
        pltpu.sync_copy(b_hbm, b_v)
        pltpu.sync_copy(sos_hbm, sos_v)

    def qstep(cc, qc):
        svec = sos_v[pl.ds(pl.multiple_of(16 * cc, 16), 16)]
        new = list(qc)
        for l in range(16):
            sk = svec[l]
            for c in range(8):
                new[c] = new[c] + sk * w_v[16 * cc + l, pl.ds(16 * c, 16)]
        return tuple(new)

    with jax.named_scope("ph1_q"):
        qcs = lax.fori_loop(0, 8, qstep,
                            tuple(b_v[pl.ds(16 * c, 16)] for c in range(8)))

    c0 = jnp.minimum(sid * _CPS, _C - _CPS)
    pltpu.sync_copy(ls_hbm.at[pl.ds(c0, _CPS), pl.ds(0, _J)], lab_v)

    def dstep(i, car):
        sv, sx = list(car[:5]), list(car[5:])
        acc = jnp.zeros((16,), jnp.float32)
        for cc in range(8):
            d = lab_v[i, pl.ds(16 * cc, 16)] - qcs[cc]
            acc = acc + d * d
        dval = jnp.sum(acc)
        sv, sx = _top5_insert((sv, sx), dval, c0 + i)
        return tuple(sv) + tuple(sx)

    with jax.named_scope("ph2_d0"):
        t5 = lax.fori_loop(0, _CPS, dstep,
                           (jnp.float32(_INF),) * 5 + (jnp.int32(_BIGI),) * 5)

    wv = jnp.full((16,), _INF, jnp.float32)
    wi = jnp.full((16,), _BIGI, jnp.int32)
    for k in range(5):
        wv = jnp.where(it == k, t5[k], wv)
        wi = jnp.where(it == k, t5[5 + k], wi)
    io16f[...] = wv
    io16i[...] = wi
    off16 = pl.multiple_of(16 * sid, 16)
    pltpu.sync_copy(io16f, cand_s.at[pl.ds(off16, 16)])
    pltpu.sync_copy(io16i, candi_s.at[pl.ds(off16, 16)])
    plsc.subcore_barrier()

    @pl.when(sid == 0)
    def _merge():
        pltpu.sync_copy(cand_s, cv_v)
        pltpu.sync_copy(candi_s, ci_v)
        vals = [cv_v[pl.ds(16 * r, 16)] for r in range(16)]
        idxs = [ci_v[pl.ds(16 * r, 16)] for r in range(16)]
        wvec = jnp.zeros((16,), jnp.int32)
        for p in range(_NBEST):
            m = vals[0]
            for r in range(1, 16):
                m = jnp.minimum(m, vals[r])
            ms = jnp.min(m)
            best = jnp.full((16,), _BIGI, jnp.int32)
            for r in range(16):
                best = jnp.minimum(best, jnp.where(vals[r] == ms, idxs[r], _BIGI))
            wid = jnp.min(best)
            vals = [jnp.where(idxs[r] == wid, _INF, vals[r]) for r in range(16)]
            wvec = jnp.where(it == p, wid, wvec)
        io16i[...] = wvec
        pltpu.sync_copy(io16i, win_s)

    plsc.subcore_barrier()
    pltpu.sync_copy(win_s, win_v)

    winvec = win_v[...]
    for m in range(_NBEST):
        r = 5 * sid + m
        n = r // 16
        t = r % 16
        cn = jnp.min(jnp.where(it == n, winvec, _BIGI))
        pltpu.sync_copy(
            ls_hbm.at[pl.ds(cn, 1), pl.ds(pl.multiple_of(_J * t, _J), _J)],
            rowbuf.at[pl.ds(m, 1), :])

    for ms in ((0, 1), (2, 3), (4,)):
        def pstep(cc, acc, ms=ms):
            lvecs = [rowbuf[m, pl.ds(pl.multiple_of(16 * cc, 16), 16)]
                     for m in ms]
            new = [list(a) for a in acc]
            for l in range(16):
                wrow = [w_v[16 * cc + l, pl.ds(16 * c, 16)] for c in range(8)]
                for j, m in enumerate(ms):
                    lk = lvecs[j][l]
                    for c in range(8):
                        new[j][c] = new[j][c] + lk * wrow[c]
            return tuple(tuple(a) for a in new)

        acc0 = tuple(tuple(b_v[pl.ds(16 * c, 16)] for c in range(8))
                     for _ in ms)
        with jax.named_scope("ph5_prow"):
            acc = lax.fori_loop(0, 8, pstep, acc0)
        for j, m in enumerate(ms):
            for c in range(8):
                prow_v[m, pl.ds(16 * c, 16)] = acc[j][c]

    pltpu.sync_copy(prow_v, pbuf_s.at[pl.ds(5 * sid, 5), :])
    plsc.subcore_barrier()
    pltpu.sync_copy(pbuf_s, p_v)

    s2parts = []
    for n in range(_NBEST):
        def mstep(t, acc, n=n):
            a = acc
            for cc in range(8):
                pch = p_v[n * 16 + t, pl.ds(16 * cc, 16)]
                a = a + pch * pch
                p_v[n * 16 + t, pl.ds(16 * cc, 16)] = 1.0 - (pch + pch)
            return a

        s2parts.append(jnp.sum(lax.fori_loop(
            0, _TL, mstep, jnp.zeros((16,), jnp.float32))))

    carry = (jnp.float32(_INF),) * _NBEST + (jnp.int32(0),) * _NBEST
    for ch in range(4):
        cs = jnp.minimum(sid * _CPS + _CHUNK * ch, _C - _CHUNK)
        pltpu.sync_copy(ls_hbm.at[pl.ds(cs, _CHUNK), :], labf_v)

        def bstep(blk, car, cs=cs):
            bv, bi = car[:_NBEST], car[_NBEST:]
            i0 = 2 * blk

            def tstep(t, accs):
                new = [list(a) for a in accs]
                toff = pl.multiple_of(_J * t, _J)
                for cc in range(8):
                    mchs = [p_v[n * 16 + t, pl.ds(16 * cc, 16)]
                            for n in range(_NBEST)]
                    for ii in range(2):
                        lch = labf_v[i0 + ii, pl.ds(toff + 16 * cc, 16)]
                        for n in range(_NBEST):
                            new[ii][n] = new[ii][n] + lch * mchs[n]
                return tuple(tuple(a) for a in new)

            accs = lax.fori_loop(
                0, _TL, tstep,
                tuple(tuple(jnp.zeros((16,), jnp.float32)
                            for _ in range(_NBEST)) for _ in range(2)))
            nbv, nbi = list(bv), list(bi)
            for ii in range(2):
                cg = cs + i0 + ii
                for n in range(_NBEST):
                    dist = s2parts[n] + jnp.sum(accs[ii][n])
                    pred = dist < nbv[n]
                    nbv[n] = jnp.where(pred, dist, nbv[n])
                    nbi[n] = jnp.where(pred, cg, nbi[n])
            return tuple(nbv) + tuple(nbi)

        with jax.named_scope("ph6_dist"):
            carry = lax.fori_loop(0, 8, bstep, carry)

    bvec = jnp.full((16,), _INF, jnp.float32)
    bivec = jnp.zeros((16,), jnp.int32)
    for n in range(_NBEST):
        bvec = jnp.where(it == n, carry[n], bvec)
        bivec = jnp.where(it == n, carry[_NBEST + n], bivec)
    io16f[...] = bvec
    io16i[...] = bivec
    pltpu.sync_copy(io16f, dmv_s.at[pl.ds(off16, 16)])
    pltpu.sync_copy(io16i, dmi_s.at[pl.ds(off16, 16)])
    plsc.subcore_barrier()

    @pl.when(jnp.logical_and(cid == 0, sid == 0))
    def _final():
        pltpu.sync_copy(dmv_s, dv_v)
        pltpu.sync_copy(dmi_s, di_v)
        runv = jnp.full((16,), _INF, jnp.float32)
        runi = jnp.zeros((16,), jnp.int32)
        for r in range(16):
            vr = dv_v[pl.ds(16 * r, 16)]
            ir = di_v[pl.ds(16 * r, 16)]
            pred = vr < runv
            runv = jnp.where(pred, vr, runv)
            runi = jnp.where(pred, ir, runi)
        runv = jnp.where(it < _NBEST, runv, _INF)
        mv = jnp.min(runv)
        lane = jnp.min(jnp.where(runv == mv, it, _BIGI))
        msk = jnp.logical_and(runv == mv, it == lane)
        label = jnp.min(jnp.where(msk, runi, _BIGI))
        lv = jnp.full((16,), label)
        out_v[pl.ds(0, 16)] = lv
        out_v[pl.ds(16, 16)] = lv
        pltpu.sync_copy(out_v, out_hbm)


def kernel(x, lens, W, b, label_seqs, sos_vec):
    mesh = plsc.VectorSubcoreMesh(core_axis_name="c", subcore_axis_name="s")
    f = pl.kernel(
        _body,
        out_type=jax.ShapeDtypeStruct((x.shape[0],), jnp.int32),
        mesh=mesh,
        compiler_params=pltpu.CompilerParams(use_tc_tiling_on_sc=False,
                                             needs_layout_passes=False),
        scratch_types=[
            pltpu.VMEM((_J, _J), jnp.float32),
            pltpu.VMEM((_J,), jnp.float32),
            pltpu.VMEM((_J,), jnp.float32),
            pltpu.VMEM((_CPS, _J), jnp.float32),
            pltpu.VMEM((_CHUNK, _TL * _J), jnp.float32),
            pltpu.VMEM((80, _J), jnp.float32),
            pltpu.VMEM((5, _J), jnp.float32),
            pltpu.VMEM((5, _J), jnp.float32),
            pltpu.VMEM((16,), jnp.int32),
            pltpu.VMEM((16,), jnp.float32),
            pltpu.VMEM((16,), jnp.int32),
            pltpu.VMEM((32,), jnp.int32),
            pltpu.VMEM((256,), jnp.float32),
            pltpu.VMEM((256,), jnp.int32),
            pltpu.VMEM((256,), jnp.float32),
            pltpu.VMEM((256,), jnp.int32),
            pltpu.VMEM_SHARED((256,), jnp.float32),
            pltpu.VMEM_SHARED((256,), jnp.int32),
            pltpu.VMEM_SHARED((16,), jnp.int32),
            pltpu.VMEM_SHARED((80, _J), jnp.float32),
            pltpu.VMEM_SHARED((256,), jnp.float32),
            pltpu.VMEM_SHARED((256,), jnp.int32),
        ],
    )
    label = f(W, b, label_seqs.reshape(_C, _TL * _J), sos_vec)
    return (label, None)


# speedup vs baseline: 1.0023x; 1.0023x over previous
import jax
import jax.numpy as jnp
from jax import lax
from jax.experimental import pallas as pl
from jax.experimental.pallas import tpu as pltpu
from jax.experimental.pallas import tpu_sc as plsc

_C = 1000
_TL = 16
_J = 128
_NBEST = 5
_CPS = 64
_CHUNK = 16
_BIGI = 1 << 30
_INF = float("inf")


def _top5_insert(slots, v, c):
    sv, sx = slots
    dup = (c == sx[0]) | (c == sx[1]) | (c == sx[2]) | (c == sx[3]) | (c == sx[4])
    lt = [jnp.logical_and(v < sv[k], jnp.logical_not(dup)) for k in range(5)]
    nv = [jnp.where(lt[0], v, sv[0])]
    nx = [jnp.where(lt[0], c, sx[0])]
    for k in range(1, 5):
        nv.append(jnp.where(lt[k], jnp.where(lt[k - 1], sv[k - 1], v), sv[k]))
        nx.append(jnp.where(lt[k], jnp.where(lt[k - 1], sx[k - 1], c), sx[k]))
    return nv, nx


def _body(w_hbm, b_hbm, ls_hbm, sos_hbm, out_hbm,
          w_v, sos_v, b_v, lab_v, labf_v, p_v, prow_v, rowbuf,
          win_v, io16f, io16i, out_v, cv_v, ci_v, dv_v, di_v,
          cand_s, candi_s, win_s, pbuf_s, dmv_s, dmi_s):
    cid = lax.axis_index("c")
    sid = lax.axis_index("s")
    it = lax.iota(jnp.int32, 16)

    with jax.named_scope("ph0_stage"):
        pltpu.sync_copy(w_hbm, w_v)
        pltpu.sync_copy(b_hbm, b_v)
        pltpu.sync_copy(sos_hbm, sos_v)

    def qstep(cc, qc):
        svec = sos_v[pl.ds(pl.multiple_of(16 * cc, 16), 16)]
        new = list(qc)
        for l in range(16):
            sk = svec[l]
            for c in range(8):
                new[c] = new[c] + sk * w_v[16 * cc + l, pl.ds(16 * c, 16)]
        return tuple(new)

    with jax.named_scope("ph1_q"):
        qcs = lax.fori_loop(0, 8, qstep,
                            tuple(b_v[pl.ds(16 * c, 16)] for c in range(8)))

    c0 = jnp.minimum(sid * _CPS, _C - _CPS)
    with jax.named_scope("ph2_dma"):
        pltpu.sync_copy(ls_hbm.at[pl.ds(c0, _CPS), pl.ds(0, _J)], lab_v)

    def dstep(i, car):
        sv, sx = list(car[:5]), list(car[5:])
        acc = jnp.zeros((16,), jnp.float32)
        for cc in range(8):
            d = lab_v[i, pl.ds(16 * cc, 16)] - qcs[cc]
            acc = acc + d * d
        dval = jnp.sum(acc)
        sv, sx = _top5_insert((sv, sx), dval, c0 + i)
        return tuple(sv) + tuple(sx)

    with jax.named_scope("ph2_d0"):
        t5 = lax.fori_loop(0, _CPS, dstep,
                           (jnp.float32(_INF),) * 5 + (jnp.int32(_BIGI),) * 5)

    wv = jnp.full((16,), _INF, jnp.float32)
    wi = jnp.full((16,), _BIGI, jnp.int32)
    for k in range(5):
        wv = jnp.where(it == k, t5[k], wv)
        wi = jnp.where(it == k, t5[5 + k], wi)
    io16f[...] = wv
    io16i[...] = wi
    off16 = pl.multiple_of(16 * sid, 16)
    with jax.named_scope("ph3_pub"):
        pltpu.sync_copy(io16f, cand_s.at[pl.ds(off16, 16)])
        pltpu.sync_copy(io16i, candi_s.at[pl.ds(off16, 16)])
    with jax.named_scope("ph3_bar1"):
        plsc.subcore_barrier()

    @pl.when(sid == 0)
    def _merge():
        pltpu.sync_copy(cand_s, cv_v)
        pltpu.sync_copy(candi_s, ci_v)
        vals = [cv_v[pl.ds(16 * r, 16)] for r in range(16)]
        idxs = [ci_v[pl.ds(16 * r, 16)] for r in range(16)]
        wvec = jnp.zeros((16,), jnp.int32)
        for p in range(_NBEST):
            m = vals[0]
            for r in range(1, 16):
                m = jnp.minimum(m, vals[r])
            ms = jnp.min(m)
            best = jnp.full((16,), _BIGI, jnp.int32)
            for r in range(16):
                best = jnp.minimum(best, jnp.where(vals[r] == ms, idxs[r], _BIGI))
            wid = jnp.min(best)
            vals = [jnp.where(idxs[r] == wid, _INF, vals[r]) for r in range(16)]
            wvec = jnp.where(it == p, wid, wvec)
        io16i[...] = wvec
        pltpu.sync_copy(io16i, win_s)

    with jax.named_scope("ph3_bar2"):
        plsc.subcore_barrier()
    pltpu.sync_copy(win_s, win_v)

    winvec = win_v[...]
    with jax.named_scope("ph5_rowdma"):
     for m in range(_NBEST):
        r = 5 * sid + m
        n = r // 16
        t = r % 16
        cn = jnp.min(jnp.where(it == n, winvec, _BIGI))
        pltpu.sync_copy(
            ls_hbm.at[pl.ds(cn, 1), pl.ds(pl.multiple_of(_J * t, _J), _J)],
            rowbuf.at[pl.ds(m, 1), :])

    for ms in ((0, 1), (2, 3), (4,)):
        def pstep(cc, acc, ms=ms):
            lvecs = [rowbuf[m, pl.ds(pl.multiple_of(16 * cc, 16), 16)]
                     for m in ms]
            new = [list(a) for a in acc]
            for l in range(16):
                wrow = [w_v[16 * cc + l, pl.ds(16 * c, 16)] for c in range(8)]
                for j, m in enumerate(ms):
                    lk = lvecs[j][l]
                    for c in range(8):
                        new[j][c] = new[j][c] + lk * wrow[c]
            return tuple(tuple(a) for a in new)

        acc0 = tuple(tuple(b_v[pl.ds(16 * c, 16)] for c in range(8))
                     for _ in ms)
        with jax.named_scope("ph5_prow"):
            acc = lax.fori_loop(0, 8, pstep, acc0)
        for j, m in enumerate(ms):
            for c in range(8):
                prow_v[m, pl.ds(16 * c, 16)] = acc[j][c]

    pltpu.sync_copy(prow_v, pbuf_s.at[pl.ds(5 * sid, 5), :])
    with jax.named_scope("ph5_bar"):
        plsc.subcore_barrier()
    with jax.named_scope("ph5_bcast"):
        pltpu.sync_copy(pbuf_s, p_v)

    s2parts = []
    for n in range(_NBEST):
        def mstep(t, acc, n=n):
            a = acc
            for cc in range(8):
                pch = p_v[n * 16 + t, pl.ds(16 * cc, 16)]
                a = a + pch * pch
                p_v[n * 16 + t, pl.ds(16 * cc, 16)] = 1.0 - (pch + pch)
            return a

        with jax.named_scope("ph5_mxf"):
            s2parts.append(jnp.sum(lax.fori_loop(
                0, _TL, mstep, jnp.zeros((16,), jnp.float32))))

    carry = (jnp.float32(_INF),) * _NBEST + (jnp.int32(0),) * _NBEST
    for ch in range(4):
        cs = jnp.minimum(sid * _CPS + _CHUNK * ch, _C - _CHUNK)
        with jax.named_scope("ph6_dma"):
            pltpu.sync_copy(ls_hbm.at[pl.ds(cs, _CHUNK), :], labf_v)

        def bstep(blk, car, cs=cs):
            bv, bi = car[:_NBEST], car[_NBEST:]
            i0 = 2 * blk

            def tstep(t, accs):
                new = [list(a) for a in accs]
                toff = pl.multiple_of(_J * t, _J)
                for cc in range(8):
                    mchs = [p_v[n * 16 + t, pl.ds(16 * cc, 16)]
                            for n in range(_NBEST)]
                    for ii in range(2):
                        lch = labf_v[i0 + ii, pl.ds(toff + 16 * cc, 16)]
                        for n in range(_NBEST):
                            new[ii][n] = new[ii][n] + lch * mchs[n]
                return tuple(tuple(a) for a in new)

            accs = lax.fori_loop(
                0, _TL, tstep,
                tuple(tuple(jnp.zeros((16,), jnp.float32)
                            for _ in range(_NBEST)) for _ in range(2)))
            nbv, nbi = list(bv), list(bi)
            for ii in range(2):
                cg = cs + i0 + ii
                for n in range(_NBEST):
                    dist = s2parts[n] + jnp.sum(accs[ii][n])
                    pred = dist < nbv[n]
                    nbv[n] = jnp.where(pred, dist, nbv[n])
                    nbi[n] = jnp.where(pred, cg, nbi[n])
            return tuple(nbv) + tuple(nbi)

        with jax.named_scope("ph6_dist"):
            carry = lax.fori_loop(0, 8, bstep, carry)

    bvec = jnp.full((16,), _INF, jnp.float32)
    bivec = jnp.zeros((16,), jnp.int32)
    for n in range(_NBEST):
        bvec = jnp.where(it == n, carry[n], bvec)
        bivec = jnp.where(it == n, carry[_NBEST + n], bivec)
    io16f[...] = bvec
    io16i[...] = bivec
    with jax.named_scope("ph7_pub"):
        pltpu.sync_copy(io16f, dmv_s.at[pl.ds(off16, 16)])
        pltpu.sync_copy(io16i, dmi_s.at[pl.ds(off16, 16)])
    with jax.named_scope("ph7_bar"):
        plsc.subcore_barrier()

    @pl.when(jnp.logical_and(cid == 0, sid == 0))
    def _final():
        pltpu.sync_copy(dmv_s, dv_v)
        pltpu.sync_copy(dmi_s, di_v)
        runv = jnp.full((16,), _INF, jnp.float32)
        runi = jnp.zeros((16,), jnp.int32)
        for r in range(16):
            vr = dv_v[pl.ds(16 * r, 16)]
            ir = di_v[pl.ds(16 * r, 16)]
            pred = vr < runv
            runv = jnp.where(pred, vr, runv)
            runi = jnp.where(pred, ir, runi)
        runv = jnp.where(it < _NBEST, runv, _INF)
        mv = jnp.min(runv)
        lane = jnp.min(jnp.where(runv == mv, it, _BIGI))
        msk = jnp.logical_and(runv == mv, it == lane)
        label = jnp.min(jnp.where(msk, runi, _BIGI))
        lv = jnp.full((16,), label)
        out_v[pl.ds(0, 16)] = lv
        out_v[pl.ds(16, 16)] = lv
        pltpu.sync_copy(out_v, out_hbm)


def kernel(x, lens, W, b, label_seqs, sos_vec):
    mesh = plsc.VectorSubcoreMesh(core_axis_name="c", subcore_axis_name="s")
    f = pl.kernel(
        _body,
        out_type=jax.ShapeDtypeStruct((x.shape[0],), jnp.int32),
        mesh=mesh,
        compiler_params=pltpu.CompilerParams(use_tc_tiling_on_sc=False,
                                             needs_layout_passes=False),
        scratch_types=[
            pltpu.VMEM((_J, _J), jnp.float32),
            pltpu.VMEM((_J,), jnp.float32),
            pltpu.VMEM((_J,), jnp.float32),
            pltpu.VMEM((_CPS, _J), jnp.float32),
            pltpu.VMEM((_CHUNK, _TL * _J), jnp.float32),
            pltpu.VMEM((80, _J), jnp.float32),
            pltpu.VMEM((5, _J), jnp.float32),
            pltpu.VMEM((5, _J), jnp.float32),
            pltpu.VMEM((16,), jnp.int32),
            pltpu.VMEM((16,), jnp.float32),
            pltpu.VMEM((16,), jnp.int32),
            pltpu.VMEM((32,), jnp.int32),
            pltpu.VMEM((256,), jnp.float32),
            pltpu.VMEM((256,), jnp.int32),
            pltpu.VMEM((256,), jnp.float32),
            pltpu.VMEM((256,), jnp.int32),
            pltpu.VMEM_SHARED((256,), jnp.float32),
            pltpu.VMEM_SHARED((256,), jnp.int32),
            pltpu.VMEM_SHARED((16,), jnp.int32),
            pltpu.VMEM_SHARED((80, _J), jnp.float32),
            pltpu.VMEM_SHARED((256,), jnp.float32),
            pltpu.VMEM_SHARED((256,), jnp.int32),
        ],
    )
    label = f(W, b, label_seqs.reshape(_C, _TL * _J), sos_vec)
    return (label, None)


# speedup vs baseline: 1.1913x; 1.1885x over previous
import jax
import jax.numpy as jnp
from jax import lax
from jax.experimental import pallas as pl
from jax.experimental.pallas import tpu as pltpu
from jax.experimental.pallas import tpu_sc as plsc

_C = 1000
_TL = 16
_J = 128
_NBEST = 5
_CPS = 64
_CHUNK = 16
_BIGI = 1 << 30
_INF = float("inf")


def _top5_insert(slots, v, c):
    sv, sx = slots
    dup = (c == sx[0]) | (c == sx[1]) | (c == sx[2]) | (c == sx[3]) | (c == sx[4])
    lt = [jnp.logical_and(v < sv[k], jnp.logical_not(dup)) for k in range(5)]
    nv = [jnp.where(lt[0], v, sv[0])]
    nx = [jnp.where(lt[0], c, sx[0])]
    for k in range(1, 5):
        nv.append(jnp.where(lt[k], jnp.where(lt[k - 1], sv[k - 1], v), sv[k]))
        nx.append(jnp.where(lt[k], jnp.where(lt[k - 1], sx[k - 1], c), sx[k]))
    return nv, nx


def _body(w_hbm, b_hbm, ls_hbm, sos_hbm, out_hbm,
          w_v, w8_v, sos_v, b_v, qp_v, qs_v, lab_v, labf_a, labf_b,
          p_v, prow_v, rowbuf, win_v, io16f, io16i, out_v,
          cv_v, ci_v, dv_v, di_v,
          cand_s, candi_s, win_s, pbuf_s, dmv_s, dmi_s, q_s,
          sem_w, sem_small, sem_lab, sem_ca, sem_cb, sem_row):
    cid = lax.axis_index("c")
    sid = lax.axis_index("s")
    it = lax.iota(jnp.int32, 16)
    c0 = jnp.minimum(sid * _CPS, _C - _CPS)

    k0 = pl.multiple_of(8 * sid, 8)
    h_w8 = pltpu.async_copy(w_hbm.at[pl.ds(k0, 8), :], w8_v, sem_small)
    h_sos = pltpu.async_copy(sos_hbm, sos_v, sem_small)
    h_b = pltpu.async_copy(b_hbm, b_v, sem_small)
    h_lab = pltpu.async_copy(ls_hbm.at[pl.ds(c0, _CPS), pl.ds(0, _J)],
                             lab_v, sem_lab)
    cs0 = jnp.minimum(c0, _C - _CHUNK)
    h_c0 = pltpu.async_copy(ls_hbm.at[pl.ds(cs0, _CHUNK), :], labf_a, sem_ca)
    h_w = pltpu.async_copy(w_hbm, w_v, sem_w)

    h_w8.wait()
    h_sos.wait()
    h_b.wait()
    svec = sos_v[pl.ds(k0, 16)]
    qp = [jnp.zeros((16,), jnp.float32) for _ in range(8)]
    for l in range(8):
        sk = svec[l]
        for c in range(8):
            qp[c] = qp[c] + sk * w8_v[l, pl.ds(16 * c, 16)]
    for c in range(8):
        qp_v[pl.ds(16 * c, 16)] = qp[c]
    off128 = pl.multiple_of(_J * sid, _J)
    pltpu.sync_copy(qp_v, q_s.at[pl.ds(off128, _J)])
    plsc.subcore_barrier()
    pltpu.sync_copy(q_s, qs_v)
    qcs = [b_v[pl.ds(16 * c, 16)] for c in range(8)]
    for r in range(16):
        for c in range(8):
            qcs[c] = qcs[c] + qs_v[pl.ds(_J * r + 16 * c, 16)]

    h_lab.wait()

    def dstep(i, car):
        sv, sx = list(car[:5]), list(car[5:])
        acc = jnp.zeros((16,), jnp.float32)
        for cc in range(8):
            d = lab_v[i, pl.ds(16 * cc, 16)] - qcs[cc]
            acc = acc + d * d
        dval = jnp.sum(acc)
        sv, sx = _top5_insert((sv, sx), dval, c0 + i)
        return tuple(sv) + tuple(sx)

    with jax.named_scope("ph2_d0"):
        t5 = lax.fori_loop(0, _CPS, dstep,
                           (jnp.float32(_INF),) * 5 + (jnp.int32(_BIGI),) * 5)

    wv = jnp.full((16,), _INF, jnp.float32)
    wi = jnp.full((16,), _BIGI, jnp.int32)
    for k in range(5):
        wv = jnp.where(it == k, t5[k], wv)
        wi = jnp.where(it == k, t5[5 + k], wi)
    io16f[...] = wv
    io16i[...] = wi
    off16 = pl.multiple_of(16 * sid, 16)
    pltpu.sync_copy(io16f, cand_s.at[pl.ds(off16, 16)])
    pltpu.sync_copy(io16i, candi_s.at[pl.ds(off16, 16)])
    plsc.subcore_barrier()

    @pl.when(sid == 0)
    def _merge():
        pltpu.sync_copy(cand_s, cv_v)
        pltpu.sync_copy(candi_s, ci_v)
        vals = [cv_v[pl.ds(16 * r, 16)] for r in range(16)]
        idxs = [ci_v[pl.ds(16 * r, 16)] for r in range(16)]
        wvec = jnp.zeros((16,), jnp.int32)
        for p in range(_NBEST):
            m = vals[0]
            for r in range(1, 16):
                m = jnp.minimum(m, vals[r])
            ms = jnp.min(m)
            best = jnp.full((16,), _BIGI, jnp.int32)
            for r in range(16):
                best = jnp.minimum(best, jnp.where(vals[r] == ms, idxs[r], _BIGI))
            wid = jnp.min(best)
            vals = [jnp.where(idxs[r] == wid, _INF, vals[r]) for r in range(16)]
            wvec = jnp.where(it == p, wid, wvec)
        io16i[...] = wvec
        pltpu.sync_copy(io16i, win_s)

    plsc.subcore_barrier()
    pltpu.sync_copy(win_s, win_v)

    winvec = win_v[...]
    hrows = []
    for m in range(_NBEST):
        r = 5 * sid + m
        n = r // 16
        t = r % 16
        cn = jnp.min(jnp.where(it == n, winvec, _BIGI))
        hrows.append(pltpu.async_copy(
            ls_hbm.at[pl.ds(cn, 1), pl.ds(pl.multiple_of(_J * t, _J), _J)],
            rowbuf.at[pl.ds(m, 1), :], sem_row))
    for h in hrows:
        h.wait()
    h_w.wait()

    for ms in ((0, 1), (2, 3), (4,)):
        def pstep(cc, acc, ms=ms):
            lvecs = [rowbuf[m, pl.ds(pl.multiple_of(16 * cc, 16), 16)]
                     for m in ms]
            new = [list(a) for a in acc]
            for l in range(16):
                wrow = [w_v[16 * cc + l, pl.ds(16 * c, 16)] for c in range(8)]
                for j, m in enumerate(ms):
                    lk = lvecs[j][l]
                    for c in range(8):
                        new[j][c] = new[j][c] + lk * wrow[c]
            return tuple(tuple(a) for a in new)

        acc0 = tuple(tuple(b_v[pl.ds(16 * c, 16)] for c in range(8))
                     for _ in ms)
        with jax.named_scope("ph5_prow"):
            acc = lax.fori_loop(0, 8, pstep, acc0)
        for j, m in enumerate(ms):
            for c in range(8):
                prow_v[m, pl.ds(16 * c, 16)] = acc[j][c]

    pltpu.sync_copy(prow_v, pbuf_s.at[pl.ds(5 * sid, 5), :])
    plsc.subcore_barrier()
    pltpu.sync_copy(pbuf_s, p_v)

    s2parts = []
    for n in range(_NBEST):
        def mstep(t, acc, n=n):
            a = acc
            for cc in range(8):
                pch = p_v[n * 16 + t, pl.ds(16 * cc, 16)]
                a = a + pch * pch
                p_v[n * 16 + t, pl.ds(16 * cc, 16)] = 1.0 - (pch + pch)
            return a

        s2parts.append(jnp.sum(lax.fori_loop(
            0, _TL, mstep, jnp.zeros((16,), jnp.float32))))

    carry = (jnp.float32(_INF),) * _NBEST + (jnp.int32(0),) * _NBEST
    bufs = (labf_a, labf_b)
    sems = (sem_ca, sem_cb)
    hs = {0: h_c0}
    for ch in range(4):
        if ch + 1 < 4:
            csn = jnp.minimum(sid * _CPS + _CHUNK * (ch + 1), _C - _CHUNK)
            hs[ch + 1] = pltpu.async_copy(
                ls_hbm.at[pl.ds(csn, _CHUNK), :],
                bufs[(ch + 1) % 2], sems[(ch + 1) % 2])
        with jax.named_scope("ph6_wait"):
            hs[ch].wait()
        cs = jnp.minimum(sid * _CPS + _CHUNK * ch, _C - _CHUNK)
        labf = bufs[ch % 2]

        def bstep(blk, car, cs=cs, labf=labf):
            bv, bi = car[:_NBEST], car[_NBEST:]
            i0 = 2 * blk

            def tstep(t, accs):
                new = [list(a) for a in accs]
                toff = pl.multiple_of(_J * t, _J)
                for cc in range(8):
                    mchs = [p_v[n * 16 + t, pl.ds(16 * cc, 16)]
                            for n in range(_NBEST)]
                    for ii in range(2):
                        lch = labf[i0 + ii, pl.ds(toff + 16 * cc, 16)]
                        for n in range(_NBEST):
                            new[ii][n] = new[ii][n] + lch * mchs[n]
                return tuple(tuple(a) for a in new)

            accs = lax.fori_loop(
                0, _TL, tstep,
                tuple(tuple(jnp.zeros((16,), jnp.float32)
                            for _ in range(_NBEST)) for _ in range(2)))
            nbv, nbi = list(bv), list(bi)
            for ii in range(2):
                cg = cs + i0 + ii
                for n in range(_NBEST):
                    dist = s2parts[n] + jnp.sum(accs[ii][n])
                    pred = dist < nbv[n]
                    nbv[n] = jnp.where(pred, dist, nbv[n])
                    nbi[n] = jnp.where(pred, cg, nbi[n])
            return tuple(nbv) + tuple(nbi)

        with jax.named_scope("ph6_dist"):
            carry = lax.fori_loop(0, 8, bstep, carry)

    bvec = jnp.full((16,), _INF, jnp.float32)
    bivec = jnp.zeros((16,), jnp.int32)
    for n in range(_NBEST):
        bvec = jnp.where(it == n, carry[n], bvec)
        bivec = jnp.where(it == n, carry[_NBEST + n], bivec)
    io16f[...] = bvec
    io16i[...] = bivec
    pltpu.sync_copy(io16f, dmv_s.at[pl.ds(off16, 16)])
    pltpu.sync_copy(io16i, dmi_s.at[pl.ds(off16, 16)])
    plsc.subcore_barrier()

    @pl.when(jnp.logical_and(cid == 0, sid == 0))
    def _final():
        pltpu.sync_copy(dmv_s, dv_v)
        pltpu.sync_copy(dmi_s, di_v)
        runv = jnp.full((16,), _INF, jnp.float32)
        runi = jnp.zeros((16,), jnp.int32)
        for r in range(16):
            vr = dv_v[pl.ds(16 * r, 16)]
            ir = di_v[pl.ds(16 * r, 16)]
            pred = vr < runv
            runv = jnp.where(pred, vr, runv)
            runi = jnp.where(pred, ir, runi)
        runv = jnp.where(it < _NBEST, runv, _INF)
        mv = jnp.min(runv)
        lane = jnp.min(jnp.where(runv == mv, it, _BIGI))
        msk = jnp.logical_and(runv == mv, it == lane)
        label = jnp.min(jnp.where(msk, runi, _BIGI))
        lv = jnp.full((16,), label)
        out_v[pl.ds(0, 16)] = lv
        out_v[pl.ds(16, 16)] = lv
        pltpu.sync_copy(out_v, out_hbm)


def kernel(x, lens, W, b, label_seqs, sos_vec):
    mesh = plsc.VectorSubcoreMesh(core_axis_name="c", subcore_axis_name="s")
    f = pl.kernel(
        _body,
        out_type=jax.ShapeDtypeStruct((x.shape[0],), jnp.int32),
        mesh=mesh,
        compiler_params=pltpu.CompilerParams(use_tc_tiling_on_sc=False,
                                             needs_layout_passes=False),
        scratch_types=[
            pltpu.VMEM((_J, _J), jnp.float32),
            pltpu.VMEM((8, _J), jnp.float32),
            pltpu.VMEM((_J,), jnp.float32),
            pltpu.VMEM((_J,), jnp.float32),
            pltpu.VMEM((_J,), jnp.float32),
            pltpu.VMEM((16 * _J,), jnp.float32),
            pltpu.VMEM((_CPS, _J), jnp.float32),
            pltpu.VMEM((_CHUNK, _TL * _J), jnp.float32),
            pltpu.VMEM((_CHUNK, _TL * _J), jnp.float32),
            pltpu.VMEM((80, _J), jnp.float32),
            pltpu.VMEM((5, _J), jnp.float32),
            pltpu.VMEM((5, _J), jnp.float32),
            pltpu.VMEM((16,), jnp.int32),
            pltpu.VMEM((16,), jnp.float32),
            pltpu.VMEM((16,), jnp.int32),
            pltpu.VMEM((32,), jnp.int32),
            pltpu.VMEM((256,), jnp.float32),
            pltpu.VMEM((256,), jnp.int32),
            pltpu.VMEM((256,), jnp.float32),
            pltpu.VMEM((256,), jnp.int32),
            pltpu.VMEM_SHARED((256,), jnp.float32),
            pltpu.VMEM_SHARED((256,), jnp.int32),
            pltpu.VMEM_SHARED((16,), jnp.int32),
            pltpu.VMEM_SHARED((80, _J), jnp.float32),
            pltpu.VMEM_SHARED((256,), jnp.float32),
            pltpu.VMEM_SHARED((256,), jnp.int32),
            pltpu.VMEM_SHARED((16 * _J,), jnp.float32),
            pltpu.SemaphoreType.DMA,
            pltpu.SemaphoreType.DMA,
            pltpu.SemaphoreType.DMA,
            pltpu.SemaphoreType.DMA,
            pltpu.SemaphoreType.DMA,
            pltpu.SemaphoreType.DMA,
        ],
    )
    label = f(W, b, label_seqs.reshape(_C, _TL * _J), sos_vec)
    return (label, None)


# speedup vs baseline: 1.2659x; 1.0626x over previous
import jax
import jax.numpy as jnp
from jax import lax
from jax.experimental import pallas as pl
from jax.experimental.pallas import tpu as pltpu
from jax.experimental.pallas import tpu_sc as plsc

_C = 1000
_TL = 16
_J = 128
_NBEST = 5
_CPS = 64
_CHUNK = 16
_BIGI = 1 << 30
_INF = float("inf")


def _top5_insert(slots, v, c):
    sv, sx = slots
    dup = (c == sx[0]) | (c == sx[1]) | (c == sx[2]) | (c == sx[3]) | (c == sx[4])
    lt = [jnp.logical_and(v < sv[k], jnp.logical_not(dup)) for k in range(5)]
    nv = [jnp.where(lt[0], v, sv[0])]
    nx = [jnp.where(lt[0], c, sx[0])]
    for k in range(1, 5):
        nv.append(jnp.where(lt[k], jnp.where(lt[k - 1], sv[k - 1], v), sv[k]))
        nx.append(jnp.where(lt[k], jnp.where(lt[k - 1], sx[k - 1], c), sx[k]))
    return nv, nx


def _body(w_hbm, b_hbm, ls_hbm, sos_hbm, out_hbm,
          w_v, w8_v, sos_v, b_v, qp_v, qs_v, lab_v, labf_a, labf_b,
          p_v, prow_v, rowbuf, win_v, io16f, io16i, out_v,
          cv_v, ci_v, dv_v, di_v,
          cand_s, candi_s, win_s, pbuf_s, dmv_s, dmi_s, q_s,
          sem_w, sem_small, sem_lab, sem_ca, sem_cb, sem_row):
    cid = lax.axis_index("c")
    sid = lax.axis_index("s")
    it = lax.iota(jnp.int32, 16)
    c0 = jnp.minimum(sid * _CPS, _C - _CPS)

    k0 = pl.multiple_of(8 * sid, 8)
    h_w8 = pltpu.async_copy(w_hbm.at[pl.ds(k0, 8), :], w8_v, sem_small)
    h_sos = pltpu.async_copy(sos_hbm, sos_v, sem_small)
    h_b = pltpu.async_copy(b_hbm, b_v, sem_small)
    h_lab = pltpu.async_copy(ls_hbm.at[pl.ds(c0, _CPS), pl.ds(0, _J)],
                             lab_v, sem_lab)
    cs0 = jnp.minimum(c0, _C - _CHUNK)
    h_c0 = pltpu.async_copy(ls_hbm.at[pl.ds(cs0, _CHUNK), :], labf_a, sem_ca)
    h_w = pltpu.async_copy(w_hbm, w_v, sem_w)

    h_w8.wait()
    h_sos.wait()
    h_b.wait()
    svec = sos_v[pl.ds(k0, 16)]
    qp = [jnp.zeros((16,), jnp.float32) for _ in range(8)]
    for l in range(8):
        sk = svec[l]
        for c in range(8):
            qp[c] = qp[c] + sk * w8_v[l, pl.ds(16 * c, 16)]
    for c in range(8):
        qp_v[pl.ds(16 * c, 16)] = qp[c]
    off128 = pl.multiple_of(_J * sid, _J)
    pltpu.sync_copy(qp_v, q_s.at[pl.ds(off128, _J)])
    plsc.subcore_barrier()
    pltpu.sync_copy(q_s, qs_v)
    qcs = [b_v[pl.ds(16 * c, 16)] for c in range(8)]
    for r in range(16):
        for c in range(8):
            qcs[c] = qcs[c] + qs_v[pl.ds(_J * r + 16 * c, 16)]

    h_lab.wait()

    def dstep(i, car):
        sv, sx = list(car[:5]), list(car[5:])
        acc = jnp.zeros((16,), jnp.float32)
        for cc in range(8):
            d = lab_v[i, pl.ds(16 * cc, 16)] - qcs[cc]
            acc = acc + d * d
        dval = jnp.sum(acc)
        sv, sx = _top5_insert((sv, sx), dval, c0 + i)
        return tuple(sv) + tuple(sx)

    with jax.named_scope("ph2_d0"):
        t5 = lax.fori_loop(0, _CPS, dstep,
                           (jnp.float32(_INF),) * 5 + (jnp.int32(_BIGI),) * 5)

    wv = jnp.full((16,), _INF, jnp.float32)
    wi = jnp.full((16,), _BIGI, jnp.int32)
    for k in range(5):
        wv = jnp.where(it == k, t5[k], wv)
        wi = jnp.where(it == k, t5[5 + k], wi)
    io16f[...] = wv
    io16i[...] = wi
    off16 = pl.multiple_of(16 * sid, 16)
    pltpu.sync_copy(io16f, cand_s.at[pl.ds(off16, 16)])
    pltpu.sync_copy(io16i, candi_s.at[pl.ds(off16, 16)])
    plsc.subcore_barrier()

    @pl.when(sid == 0)
    def _merge():
        pltpu.sync_copy(cand_s, cv_v)
        pltpu.sync_copy(candi_s, ci_v)
        vals = [cv_v[pl.ds(16 * r, 16)] for r in range(16)]
        idxs = [ci_v[pl.ds(16 * r, 16)] for r in range(16)]
        wvec = jnp.zeros((16,), jnp.int32)
        for p in range(_NBEST):
            m = vals[0]
            for r in range(1, 16):
                m = jnp.minimum(m, vals[r])
            ms = jnp.min(m)
            best = jnp.full((16,), _BIGI, jnp.int32)
            for r in range(16):
                best = jnp.minimum(best, jnp.where(vals[r] == ms, idxs[r], _BIGI))
            wid = jnp.min(best)
            vals = [jnp.where(idxs[r] == wid, _INF, vals[r]) for r in range(16)]
            wvec = jnp.where(it == p, wid, wvec)
        io16i[...] = wvec
        pltpu.sync_copy(io16i, win_s)

    plsc.subcore_barrier()
    pltpu.sync_copy(win_s, win_v)

    winvec = win_v[...]
    hrows = []
    for m in range(_NBEST):
        r = 5 * sid + m
        n = r // 16
        t = r % 16
        cn = jnp.min(jnp.where(it == n, winvec, _BIGI))
        hrows.append(pltpu.async_copy(
            ls_hbm.at[pl.ds(cn, 1), pl.ds(pl.multiple_of(_J * t, _J), _J)],
            rowbuf.at[pl.ds(m, 1), :], sem_row))
    for h in hrows:
        h.wait()
    h_w.wait()

    for m in range(_NBEST):
        def pstep(cc, acc, m=m):
            lvec = rowbuf[m, pl.ds(pl.multiple_of(16 * cc, 16), 16)]
            new = list(acc)
            for l in range(16):
                lk = lvec[l]
                for c in range(8):
                    new[c] = new[c] + lk * w_v[16 * cc + l, pl.ds(16 * c, 16)]
            return tuple(new)

        acc0 = tuple(b_v[pl.ds(16 * c, 16)] for c in range(8))
        with jax.named_scope("ph5_prow"):
            acc = lax.fori_loop(0, 8, pstep, acc0)
        for c in range(8):
            prow_v[m, pl.ds(16 * c, 16)] = acc[c]

    pltpu.sync_copy(prow_v, pbuf_s.at[pl.ds(5 * sid, 5), :])
    plsc.subcore_barrier()
    pltpu.sync_copy(pbuf_s, p_v)

    s2parts = []
    for n in range(_NBEST):
        def mstep(t, acc, n=n):
            a = acc
            for cc in range(8):
                pch = p_v[n * 16 + t, pl.ds(16 * cc, 16)]
                a = a + pch * pch
                p_v[n * 16 + t, pl.ds(16 * cc, 16)] = 1.0 - (pch + pch)
            return a

        s2parts.append(jnp.sum(lax.fori_loop(
            0, _TL, mstep, jnp.zeros((16,), jnp.float32))))

    carry = (jnp.float32(_INF),) * _NBEST + (jnp.int32(0),) * _NBEST
    bufs = (labf_a, labf_b)
    sems = (sem_ca, sem_cb)
    hs = {0: h_c0}
    for ch in range(4):
        if ch + 1 < 4:
            csn = jnp.minimum(sid * _CPS + _CHUNK * (ch + 1), _C - _CHUNK)
            hs[ch + 1] = pltpu.async_copy(
                ls_hbm.at[pl.ds(csn, _CHUNK), :],
                bufs[(ch + 1) % 2], sems[(ch + 1) % 2])
        with jax.named_scope("ph6_wait"):
            hs[ch].wait()
        cs = jnp.minimum(sid * _CPS + _CHUNK * ch, _C - _CHUNK)
        labf = bufs[ch % 2]

        def bstep(blk, car, cs=cs, labf=labf):
            bv, bi = car[:_NBEST], car[_NBEST:]
            i0 = 2 * blk

            def tstep(t, accs):
                new = [list(a) for a in accs]
                toff = pl.multiple_of(_J * t, _J)
                for cc in range(8):
                    mchs = [p_v[n * 16 + t, pl.ds(16 * cc, 16)]
                            for n in range(_NBEST)]
                    for ii in range(2):
                        lch = labf[i0 + ii, pl.ds(toff + 16 * cc, 16)]
                        for n in range(_NBEST):
                            new[ii][n] = new[ii][n] + lch * mchs[n]
                return tuple(tuple(a) for a in new)

            accs = lax.fori_loop(
                0, _TL, tstep,
                tuple(tuple(jnp.zeros((16,), jnp.float32)
                            for _ in range(_NBEST)) for _ in range(2)))
            nbv, nbi = list(bv), list(bi)
            for ii in range(2):
                cg = cs + i0 + ii
                for n in range(_NBEST):
                    dist = s2parts[n] + jnp.sum(accs[ii][n])
                    pred = dist < nbv[n]
                    nbv[n] = jnp.where(pred, dist, nbv[n])
                    nbi[n] = jnp.where(pred, cg, nbi[n])
            return tuple(nbv) + tuple(nbi)

        with jax.named_scope("ph6_dist"):
            carry = lax.fori_loop(0, 8, bstep, carry)

    bvec = jnp.full((16,), _INF, jnp.float32)
    bivec = jnp.zeros((16,), jnp.int32)
    for n in range(_NBEST):
        bvec = jnp.where(it == n, carry[n], bvec)
        bivec = jnp.where(it == n, carry[_NBEST + n], bivec)
    io16f[...] = bvec
    io16i[...] = bivec
    pltpu.sync_copy(io16f, dmv_s.at[pl.ds(off16, 16)])
    pltpu.sync_copy(io16i, dmi_s.at[pl.ds(off16, 16)])
    plsc.subcore_barrier()

    @pl.when(jnp.logical_and(cid == 0, sid == 0))
    def _final():
        pltpu.sync_copy(dmv_s, dv_v)
        pltpu.sync_copy(dmi_s, di_v)
        runv = jnp.full((16,), _INF, jnp.float32)
        runi = jnp.zeros((16,), jnp.int32)
        for r in range(16):
            vr = dv_v[pl.ds(16 * r, 16)]
            ir = di_v[pl.ds(16 * r, 16)]
            pred = vr < runv
            runv = jnp.where(pred, vr, runv)
            runi = jnp.where(pred, ir, runi)
        runv = jnp.where(it < _NBEST, runv, _INF)
        mv = jnp.min(runv)
        lane = jnp.min(jnp.where(runv == mv, it, _BIGI))
        msk = jnp.logical_and(runv == mv, it == lane)
        label = jnp.min(jnp.where(msk, runi, _BIGI))
        lv = jnp.full((16,), label)
        out_v[pl.ds(0, 16)] = lv
        out_v[pl.ds(16, 16)] = lv
        pltpu.sync_copy(out_v, out_hbm)


def kernel(x, lens, W, b, label_seqs, sos_vec):
    mesh = plsc.VectorSubcoreMesh(core_axis_name="c", subcore_axis_name="s")
    f = pl.kernel(
        _body,
        out_type=jax.ShapeDtypeStruct((x.shape[0],), jnp.int32),
        mesh=mesh,
        compiler_params=pltpu.CompilerParams(use_tc_tiling_on_sc=False,
                                             needs_layout_passes=False),
        scratch_types=[
            pltpu.VMEM((_J, _J), jnp.float32),
            pltpu.VMEM((8, _J), jnp.float32),
            pltpu.VMEM((_J,), jnp.float32),
            pltpu.VMEM((_J,), jnp.float32),
            pltpu.VMEM((_J,), jnp.float32),
            pltpu.VMEM((16 * _J,), jnp.float32),
            pltpu.VMEM((_CPS, _J), jnp.float32),
            pltpu.VMEM((_CHUNK, _TL * _J), jnp.float32),
            pltpu.VMEM((_CHUNK, _TL * _J), jnp.float32),
            pltpu.VMEM((80, _J), jnp.float32),
            pltpu.VMEM((5, _J), jnp.float32),
            pltpu.VMEM((5, _J), jnp.float32),
            pltpu.VMEM((16,), jnp.int32),
            pltpu.VMEM((16,), jnp.float32),
            pltpu.VMEM((16,), jnp.int32),
            pltpu.VMEM((32,), jnp.int32),
            pltpu.VMEM((256,), jnp.float32),
            pltpu.VMEM((256,), jnp.int32),
            pltpu.VMEM((256,), jnp.float32),
            pltpu.VMEM((256,), jnp.int32),
            pltpu.VMEM_SHARED((256,), jnp.float32),
            pltpu.VMEM_SHARED((256,), jnp.int32),
            pltpu.VMEM_SHARED((16,), jnp.int32),
            pltpu.VMEM_SHARED((80, _J), jnp.float32),
            pltpu.VMEM_SHARED((256,), jnp.float32),
            pltpu.VMEM_SHARED((256,), jnp.int32),
            pltpu.VMEM_SHARED((16 * _J,), jnp.float32),
            pltpu.SemaphoreType.DMA,
            pltpu.SemaphoreType.DMA,
            pltpu.SemaphoreType.DMA,
            pltpu.SemaphoreType.DMA,
            pltpu.SemaphoreType.DMA,
            pltpu.SemaphoreType.DMA,
        ],
    )
    label = f(W, b, label_seqs.reshape(_C, _TL * _J), sos_vec)
    return (label, None)


# speedup vs baseline: 1.3876x; 1.0962x over previous
import jax
import jax.numpy as jnp
from jax import lax
from jax.experimental import pallas as pl
from jax.experimental.pallas import tpu as pltpu
from jax.experimental.pallas import tpu_sc as plsc

_C = 1000
_TL = 16
_J = 128
_NBEST = 5
_CPS = 64
_CHUNK = 16
_BIGI = 1 << 30
_INF = float("inf")


def _top5_insert(slots, v, c):
    sv, sx = slots
    dup = (c == sx[0]) | (c == sx[1]) | (c == sx[2]) | (c == sx[3]) | (c == sx[4])
    lt = [jnp.logical_and(v < sv[k], jnp.logical_not(dup)) for k in range(5)]
    nv = [jnp.where(lt[0], v, sv[0])]
    nx = [jnp.where(lt[0], c, sx[0])]
    for k in range(1, 5):
        nv.append(jnp.where(lt[k], jnp.where(lt[k - 1], sv[k - 1], v), sv[k]))
        nx.append(jnp.where(lt[k], jnp.where(lt[k - 1], sx[k - 1], c), sx[k]))
    return nv, nx


def _body(w_hbm, b_hbm, ls_hbm, sos_hbm, pv_hbm, pi_hbm,
          w_v, w8_v, sos_v, b_v, qp_v, qs_v, lab_v, labf_a, labf_b,
          p_v, prow_v, rowbuf, win_v, io16f, io16i, out_v,
          cv_v, ci_v, dv_v, di_v,
          cand_s, candi_s, win_s, pbuf_s, dmv_s, dmi_s, q_s,
          sem_w, sem_small, sem_lab, sem_ca, sem_cb, sem_row):
    cid = lax.axis_index("c")
    sid = lax.axis_index("s")
    it = lax.iota(jnp.int32, 16)
    c0 = jnp.minimum(sid * _CPS, _C - _CPS)
    base6 = jnp.minimum(512 * cid + 32 * sid, _C - 32)

    k0 = pl.multiple_of(8 * sid, 8)
    h_w8 = pltpu.async_copy(w_hbm.at[pl.ds(k0, 8), :], w8_v, sem_small)
    h_sos = pltpu.async_copy(sos_hbm, sos_v, sem_small)
    h_b = pltpu.async_copy(b_hbm, b_v, sem_small)
    h_lab = pltpu.async_copy(ls_hbm.at[pl.ds(c0, _CPS), pl.ds(0, _J)],
                             lab_v, sem_lab)
    cs0 = jnp.minimum(base6, _C - _CHUNK)
    h_c0 = pltpu.async_copy(ls_hbm.at[pl.ds(cs0, _CHUNK), :], labf_a, sem_ca)
    h_w = pltpu.async_copy(w_hbm, w_v, sem_w)

    h_w8.wait()
    h_sos.wait()
    h_b.wait()
    svec = sos_v[pl.ds(k0, 16)]
    qp = [jnp.zeros((16,), jnp.float32) for _ in range(8)]
    for l in range(8):
        sk = svec[l]
        for c in range(8):
            qp[c] = qp[c] + sk * w8_v[l, pl.ds(16 * c, 16)]
    for c in range(8):
        qp_v[pl.ds(16 * c, 16)] = qp[c]
    off128 = pl.multiple_of(_J * sid, _J)
    pltpu.sync_copy(qp_v, q_s.at[pl.ds(off128, _J)])
    plsc.subcore_barrier()
    pltpu.sync_copy(q_s, qs_v)
    qcs = [b_v[pl.ds(16 * c, 16)] for c in range(8)]
    for r in range(16):
        for c in range(8):
            qcs[c] = qcs[c] + qs_v[pl.ds(_J * r + 16 * c, 16)]

    h_lab.wait()

    def dstep(i, car):
        sv, sx = list(car[:5]), list(car[5:])
        acc = jnp.zeros((16,), jnp.float32)
        for cc in range(8):
            d = lab_v[i, pl.ds(16 * cc, 16)] - qcs[cc]
            acc = acc + d * d
        dval = jnp.sum(acc)
        sv, sx = _top5_insert((sv, sx), dval, c0 + i)
        return tuple(sv) + tuple(sx)

    with jax.named_scope("ph2_d0"):
        t5 = lax.fori_loop(0, _CPS, dstep,
                           (jnp.float32(_INF),) * 5 + (jnp.int32(_BIGI),) * 5)

    wv = jnp.full((16,), _INF, jnp.float32)
    wi = jnp.full((16,), _BIGI, jnp.int32)
    for k in range(5):
        wv = jnp.where(it == k, t5[k], wv)
        wi = jnp.where(it == k, t5[5 + k], wi)
    io16f[...] = wv
    io16i[...] = wi
    off16 = pl.multiple_of(16 * sid, 16)
    pltpu.sync_copy(io16f, cand_s.at[pl.ds(off16, 16)])
    pltpu.sync_copy(io16i, candi_s.at[pl.ds(off16, 16)])
    plsc.subcore_barrier()

    @pl.when(sid == 0)
    def _merge():
        pltpu.sync_copy(cand_s, cv_v)
        pltpu.sync_copy(candi_s, ci_v)
        vals = [cv_v[pl.ds(16 * r, 16)] for r in range(16)]
        idxs = [ci_v[pl.ds(16 * r, 16)] for r in range(16)]
        wvec = jnp.zeros((16,), jnp.int32)
        for p in range(_NBEST):
            m = vals[0]
            for r in range(1, 16):
                m = jnp.minimum(m, vals[r])
            ms = jnp.min(m)
            best = jnp.full((16,), _BIGI, jnp.int32)
            for r in range(16):
                best = jnp.minimum(best, jnp.where(vals[r] == ms, idxs[r], _BIGI))
            wid = jnp.min(best)
            vals = [jnp.where(idxs[r] == wid, _INF, vals[r]) for r in range(16)]
            wvec = jnp.where(it == p, wid, wvec)
        io16i[...] = wvec
        pltpu.sync_copy(io16i, win_s)

    plsc.subcore_barrier()
    pltpu.sync_copy(win_s, win_v)

    winvec = win_v[...]
    hrows = []
    for m in range(_NBEST):
        r = 5 * sid + m
        n = r // 16
        t = r % 16
        cn = jnp.min(jnp.where(it == n, winvec, _BIGI))
        hrows.append(pltpu.async_copy(
            ls_hbm.at[pl.ds(cn, 1), pl.ds(pl.multiple_of(_J * t, _J), _J)],
            rowbuf.at[pl.ds(m, 1), :], sem_row))
    for h in hrows:
        h.wait()
    h_w.wait()

    for m in range(_NBEST):
        def pstep(cc, acc, m=m):
            lvec = rowbuf[m, pl.ds(pl.multiple_of(16 * cc, 16), 16)]
            new = list(acc)
            for l in range(16):
                lk = lvec[l]
                for c in range(8):
                    new[c] = new[c] + lk * w_v[16 * cc + l, pl.ds(16 * c, 16)]
            return tuple(new)

        acc0 = tuple(b_v[pl.ds(16 * c, 16)] for c in range(8))
        with jax.named_scope("ph5_prow"):
            acc = lax.fori_loop(0, 8, pstep, acc0)
        for c in range(8):
            prow_v[m, pl.ds(16 * c, 16)] = acc[c]

    pltpu.sync_copy(prow_v, pbuf_s.at[pl.ds(5 * sid, 5), :])
    plsc.subcore_barrier()
    pltpu.sync_copy(pbuf_s, p_v)

    s2parts = []
    for n in range(_NBEST):
        def mstep(t, acc, n=n):
            a = acc
            for cc in range(8):
                pch = p_v[n * 16 + t, pl.ds(16 * cc, 16)]
                a = a + pch * pch
                p_v[n * 16 + t, pl.ds(16 * cc, 16)] = 1.0 - (pch + pch)
            return a

        s2parts.append(jnp.sum(lax.fori_loop(
            0, _TL, mstep, jnp.zeros((16,), jnp.float32))))

    carry = (jnp.float32(_INF),) * _NBEST + (jnp.int32(0),) * _NBEST
    bufs = (labf_a, labf_b)
    sems = (sem_ca, sem_cb)
    hs = {0: h_c0}
    for ch in range(2):
        if ch + 1 < 2:
            csn = jnp.minimum(base6 + _CHUNK * (ch + 1), _C - _CHUNK)
            hs[ch + 1] = pltpu.async_copy(
                ls_hbm.at[pl.ds(csn, _CHUNK), :],
                bufs[(ch + 1) % 2], sems[(ch + 1) % 2])
        with jax.named_scope("ph6_wait"):
            hs[ch].wait()
        cs = jnp.minimum(base6 + _CHUNK * ch, _C - _CHUNK)
        labf = bufs[ch % 2]

        def bstep(blk, car, cs=cs, labf=labf):
            bv, bi = car[:_NBEST], car[_NBEST:]
            i0 = 2 * blk

            def tstep(t, accs):
                new = [list(a) for a in accs]
                toff = pl.multiple_of(_J * t, _J)
                for cc in range(8):
                    mchs = [p_v[n * 16 + t, pl.ds(16 * cc, 16)]
                            for n in range(_NBEST)]
                    for ii in range(2):
                        lch = labf[i0 + ii, pl.ds(toff + 16 * cc, 16)]
                        for n in range(_NBEST):
                            new[ii][n] = new[ii][n] + lch * mchs[n]
                return tuple(tuple(a) for a in new)

            accs = lax.fori_loop(
                0, _TL, tstep,
                tuple(tuple(jnp.zeros((16,), jnp.float32)
                            for _ in range(_NBEST)) for _ in range(2)))
            nbv, nbi = list(bv), list(bi)
            for ii in range(2):
                cg = cs + i0 + ii
                for n in range(_NBEST):
                    dist = s2parts[n] + jnp.sum(accs[ii][n])
                    pred = dist < nbv[n]
                    nbv[n] = jnp.where(pred, dist, nbv[n])
                    nbi[n] = jnp.where(pred, cg, nbi[n])
            return tuple(nbv) + tuple(nbi)

        with jax.named_scope("ph6_dist"):
            carry = lax.fori_loop(0, 8, bstep, carry)

    bvec = jnp.full((16,), _INF, jnp.float32)
    bivec = jnp.zeros((16,), jnp.int32)
    for n in range(_NBEST):
        bvec = jnp.where(it == n, carry[n], bvec)
        bivec = jnp.where(it == n, carry[_NBEST + n], bivec)
    io16f[...] = bvec
    io16i[...] = bivec
    pltpu.sync_copy(io16f, dmv_s.at[pl.ds(off16, 16)])
    pltpu.sync_copy(io16i, dmi_s.at[pl.ds(off16, 16)])
    plsc.subcore_barrier()

    @pl.when(sid == 0)
    def _final():
        pltpu.sync_copy(dmv_s, dv_v)
        pltpu.sync_copy(dmi_s, di_v)
        runv = jnp.full((16,), _INF, jnp.float32)
        runi = jnp.zeros((16,), jnp.int32)
        for r in range(16):
            vr = dv_v[pl.ds(16 * r, 16)]
            ir = di_v[pl.ds(16 * r, 16)]
            pred = vr < runv
            runv = jnp.where(pred, vr, runv)
            runi = jnp.where(pred, ir, runi)
        io16f[...] = runv
        io16i[...] = runi
        offc = pl.multiple_of(16 * cid, 16)
        pltpu.sync_copy(io16f, pv_hbm.at[pl.ds(offc, 16)])
        pltpu.sync_copy(io16i, pi_hbm.at[pl.ds(offc, 16)])


def _merge_tc(v_ref, i_ref, o_ref):
    v = v_ref[...]
    idx = i_ref[...]
    lane = lax.broadcasted_iota(jnp.int32, (1, 16), 1)
    pred = v[1:2, :] < v[0:1, :]
    runv = jnp.where(pred, v[1:2, :], v[0:1, :])
    runi = jnp.where(pred, idx[1:2, :], idx[0:1, :])
    runv = jnp.where(lane < _NBEST, runv, jnp.float32(_INF))
    mv = jnp.min(runv)
    lsel = jnp.min(jnp.where(runv == mv, lane, _BIGI))
    sel = jnp.logical_and(runv == mv, lane == lsel)
    label = jnp.min(jnp.where(sel, runi, _BIGI))
    o_ref[...] = jnp.full((1, 32), label, jnp.int32)


def kernel(x, lens, W, b, label_seqs, sos_vec):
    mesh = plsc.VectorSubcoreMesh(core_axis_name="c", subcore_axis_name="s")
    f = pl.kernel(
        _body,
        out_type=(jax.ShapeDtypeStruct((32,), jnp.float32),
                  jax.ShapeDtypeStruct((32,), jnp.int32)),
        mesh=mesh,
        compiler_params=pltpu.CompilerParams(use_tc_tiling_on_sc=False,
                                             needs_layout_passes=False),
        scratch_types=[
            pltpu.VMEM((_J, _J), jnp.float32),
            pltpu.VMEM((8, _J), jnp.float32),
            pltpu.VMEM((_J,), jnp.float32),
            pltpu.VMEM((_J,), jnp.float32),
            pltpu.VMEM((_J,), jnp.float32),
            pltpu.VMEM((16 * _J,), jnp.float32),
            pltpu.VMEM((_CPS, _J), jnp.float32),
            pltpu.VMEM((_CHUNK, _TL * _J), jnp.float32),
            pltpu.VMEM((_CHUNK, _TL * _J), jnp.float32),
            pltpu.VMEM((80, _J), jnp.float32),
            pltpu.VMEM((5, _J), jnp.float32),
            pltpu.VMEM((5, _J), jnp.float32),
            pltpu.VMEM((16,), jnp.int32),
            pltpu.VMEM((16,), jnp.float32),
            pltpu.VMEM((16,), jnp.int32),
            pltpu.VMEM((32,), jnp.int32),
            pltpu.VMEM((256,), jnp.float32),
            pltpu.VMEM((256,), jnp.int32),
            pltpu.VMEM((256,), jnp.float32),
            pltpu.VMEM((256,), jnp.int32),
            pltpu.VMEM_SHARED((256,), jnp.float32),
            pltpu.VMEM_SHARED((256,), jnp.int32),
            pltpu.VMEM_SHARED((16,), jnp.int32),
            pltpu.VMEM_SHARED((80, _J), jnp.float32),
            pltpu.VMEM_SHARED((256,), jnp.float32),
            pltpu.VMEM_SHARED((256,), jnp.int32),
            pltpu.VMEM_SHARED((16 * _J,), jnp.float32),
            pltpu.SemaphoreType.DMA,
            pltpu.SemaphoreType.DMA,
            pltpu.SemaphoreType.DMA,
            pltpu.SemaphoreType.DMA,
            pltpu.SemaphoreType.DMA,
            pltpu.SemaphoreType.DMA,
        ],
    )
    pv, pi = f(W, b, label_seqs.reshape(_C, _TL * _J), sos_vec)
    lab = pl.pallas_call(
        _merge_tc,
        out_shape=jax.ShapeDtypeStruct((1, 32), jnp.int32),
    )(pv.reshape(2, 16), pi.reshape(2, 16))
    return (lab.reshape(x.shape[0]), None)
